# baseline (speedup 1.0000x reference)
import functools

import jax
import jax.numpy as jnp
from jax import lax
from jax.experimental import pallas as pl
from jax.experimental.pallas import tpu as pltpu
from jax.experimental.pallas import tpu_sc as plsc

N = 10000
E = 320000
D = 128
NC, NS = 2, 16
NWK = NC * NS
GW = 80
CH = 80
MW = 144
BN = 2000
BE = 2560
SLABS = ((0, 120320), (120320, 158720), (279040, 40960))

f32 = jnp.float32
i32 = jnp.int32
u32 = jnp.uint32

_sc_mesh = plsc.VectorSubcoreMesh(
    core_axis_name="c", subcore_axis_name="s", num_cores=NC, num_subcores=NS)
_sc_params = pltpu.CompilerParams(use_tc_tiling_on_sc=False)


def _pack2(lo_f, hi_f):
    lo_u = lax.bitcast_convert_type(lo_f, u32) + u32(0x8000)
    hi_u = lax.bitcast_convert_type(hi_f, u32) + u32(0x8000)
    w = lax.bitwise_or(lax.shift_right_logical(lo_u, u32(16)),
                       lax.bitwise_and(hi_u, u32(0xFFFF0000)))
    return lax.bitcast_convert_type(w, i32)


def _unpack2(w_i32):
    w = lax.bitcast_convert_type(w_i32, u32)
    lo = lax.bitcast_convert_type(lax.shift_left(w, u32(16)), f32)
    hi = lax.bitcast_convert_type(lax.bitwise_and(w, u32(0xFFFF0000)), f32)
    return lo, hi


def _tables_body(x_ref, d_ref, wal_ref, wah_ref, wcl_ref, wch_ref,
                 bl_ref, bh_ref, ts_ref, td_ref):
    xb = x_ref[...]
    db = d_ref[...]
    dp = _pack2(db[:, :8], db[:, 8:16])
    z = jnp.zeros((xb.shape[0], 8), i32)
    ps_lo = jnp.dot(xb, wal_ref[...], preferred_element_type=f32) + bl_ref[...]
    ps_hi = jnp.dot(xb, wah_ref[...], preferred_element_type=f32) + bh_ref[...]
    pd_lo = jnp.dot(xb, wcl_ref[...], preferred_element_type=f32)
    pd_hi = jnp.dot(xb, wch_ref[...], preferred_element_type=f32)
    ts_ref[...] = jnp.concatenate([_pack2(ps_lo, ps_hi), dp, z], axis=1)
    td_ref[...] = jnp.concatenate([_pack2(pd_lo, pd_hi), dp, z], axis=1)


def _build_tables(x, desc16, wal, wah, wcl, wch, bl, bh):
    full = lambda a, b: pl.BlockSpec((a, b), lambda i: (0, 0))
    return pl.pallas_call(
        _tables_body,
        grid=(N // BN,),
        in_specs=[pl.BlockSpec((BN, D), lambda i: (i, 0)),
                  pl.BlockSpec((BN, 16), lambda i: (i, 0)),
                  full(D, 64), full(D, 64), full(D, 64), full(D, 64),
                  full(1, 64), full(1, 64)],
        out_specs=[pl.BlockSpec((BN, 80), lambda i: (i, 0)),
                   pl.BlockSpec((BN, 80), lambda i: (i, 0))],
        out_shape=[jax.ShapeDtypeStruct((N, 80), i32),
                   jax.ShapeDtypeStruct((N, 80), i32)],
    )(x, desc16, wal, wah, wcl, wch, bl, bh)


def _gather_stage(ts, td, src, dst):
    SLAB = src.shape[0]
    EPW = SLAB // NWK
    STEPS = EPW // GW

    @functools.partial(
        pl.kernel,
        out_type=(jax.ShapeDtypeStruct((SLAB, D), i32),
                  jax.ShapeDtypeStruct((SLAB, D), i32)),
        mesh=_sc_mesh,
        scratch_types=[pltpu.VMEM((2, GW), i32),
                       pltpu.VMEM((2, GW), i32),
                       pltpu.VMEM((2, GW, 80), i32),
                       pltpu.VMEM((2, GW, 80), i32),
                       pltpu.SemaphoreType.DMA,
                       pltpu.SemaphoreType.DMA,
                       pltpu.SemaphoreType.DMA,
                       pltpu.SemaphoreType.DMA,
                       pltpu.SemaphoreType.DMA,
                       pltpu.SemaphoreType.DMA],
        compiler_params=_sc_params)
    def gk(ts_hbm, td_hbm, src_hbm, dst_hbm,
           gs_hbm, gd_hbm,
           is_v, id_v, gs_v, gd_v,
           semi0, semi1, semg0, semg1, semo0, semo1):
        wid = lax.axis_index("s") * NC + lax.axis_index("c")
        base = wid * EPW
        semi = (semi0, semi1)
        semg = (semg0, semg1)
        semo = (semo0, semo1)

        def issue_idx(w, b):
            off = base + w * GW
            pltpu.async_copy(src_hbm.at[pl.ds(off, GW)], is_v.at[b], semi[b])
            pltpu.async_copy(dst_hbm.at[pl.ds(off, GW)], id_v.at[b], semi[b])

        def wait_idx(w, b):
            off = base + w * GW
            pltpu.make_async_copy(src_hbm.at[pl.ds(off, GW)], is_v.at[b], semi[b]).wait()
            pltpu.make_async_copy(dst_hbm.at[pl.ds(off, GW)], id_v.at[b], semi[b]).wait()

        def issue_gather(b):
            pltpu.async_copy(ts_hbm.at[is_v.at[b]], gs_v.at[b], semg[b])
            pltpu.async_copy(td_hbm.at[id_v.at[b]], gd_v.at[b], semg[b])

        def wait_gather(b):
            pltpu.make_async_copy(ts_hbm.at[is_v.at[b]], gs_v.at[b], semg[b]).wait()
            pltpu.make_async_copy(td_hbm.at[id_v.at[b]], gd_v.at[b], semg[b]).wait()

        def issue_out(w, b):
            off = base + w * GW
            pltpu.async_copy(gs_v.at[b], gs_hbm.at[pl.ds(off, GW), pl.ds(0, 80)], semo[b])
            pltpu.async_copy(gd_v.at[b], gd_hbm.at[pl.ds(off, GW), pl.ds(0, 80)], semo[b])

        def wait_out(w, b):
            off = base + w * GW
            pltpu.make_async_copy(gs_v.at[b], gs_hbm.at[pl.ds(off, GW), pl.ds(0, 80)], semo[b]).wait()
            pltpu.make_async_copy(gd_v.at[b], gd_hbm.at[pl.ds(off, GW), pl.ds(0, 80)], semo[b]).wait()

        issue_idx(0, 0)
        issue_idx(1, 1)
        wait_idx(0, 0)
        issue_gather(0)
        wait_idx(1, 1)
        issue_gather(1)

        @pl.loop(0, (STEPS + 1) // 2)
        def _(g):
            for b in (0, 1):
                w = 2 * g + b

                @pl.when(w < STEPS)
                def _():
                    wait_gather(b)
                    issue_out(w, b)

                    @pl.when(w + 2 < STEPS)
                    def _():
                        issue_idx(w + 2, b)
                        wait_out(w, b)
                        wait_idx(w + 2, b)
                        issue_gather(b)

        wait_out(STEPS - 2, (STEPS - 2) % 2)
        wait_out(STEPS - 1, (STEPS - 1) % 2)

    return gk(ts, td, src, dst)


def _edge_body(gs_ref, gd_ref, w1_ref, b1_ref, w2_ref,
               b2_ref, wm_ref, wr1_ref, br1_ref,
               wrel_ref, brel_ref, msg_ref, ec_ref):
    s_lo, s_hi = _unpack2(gs_ref[...])
    d_lo, d_hi = _unpack2(gd_ref[...])
    dcat = jnp.concatenate([s_lo[:, 64:72], s_hi[:, 64:72],
                            d_lo[:, 64:72], d_hi[:, 64:72]], axis=1)
    h = jax.nn.relu(jnp.dot(dcat, w1_ref[...], preferred_element_type=f32)
                    + b1_ref[...])
    e = jax.nn.relu(jnp.dot(h, w2_ref[...], preferred_element_type=f32)
                    + b2_ref[...])
    p = jnp.concatenate([s_lo[:, :64] + d_lo[:, :64],
                         s_hi[:, :64] + d_hi[:, :64]], axis=1)
    m = jax.nn.relu(p + jnp.dot(e, wm_ref[...], preferred_element_type=f32))
    msg_ref[...] = m
    hr = jax.nn.relu(jnp.dot(m, wr1_ref[...], preferred_element_type=f32)
                     + br1_ref[...])
    ec_ref[...] = jnp.dot(hr, wrel_ref[...], preferred_element_type=f32) \
        + brel_ref[...]


def _edge_stage(gs, gd, w1p, be1r, We2, be2r, wg1m, Wr1, br1r,
                Wrel, brelr):
    SLAB = gs.shape[0]
    full = lambda a, b: pl.BlockSpec((a, b), lambda i: (0, 0))
    return pl.pallas_call(
        _edge_body,
        grid=(SLAB // BE,),
        in_specs=[pl.BlockSpec((BE, D), lambda i: (i, 0)),
                  pl.BlockSpec((BE, D), lambda i: (i, 0)),
                  full(32, 64), full(1, 64),
                  full(64, D), full(1, D),
                  full(D, D),
                  full(D, D), full(1, D),
                  full(D, 26), full(1, 26)],
        out_specs=[pl.BlockSpec((BE, D), lambda i: (i, 0)),
                   pl.BlockSpec((BE, 26), lambda i: (i, 0))],
        out_shape=[jax.ShapeDtypeStruct((SLAB, D), f32),
                   jax.ShapeDtypeStruct((SLAB, 26), f32)],
    )(gs, gd, w1p, be1r, We2, be2r, wg1m, Wr1, br1r, Wrel, brelr)


def _scatter_stage(msg, dst):
    SLAB = msg.shape[0]
    EPW = SLAB // NWK
    STEPS = EPW // CH
    STR = N // NS
    ZR = 125

    @functools.partial(
        pl.kernel,
        out_type=(jax.ShapeDtypeStruct((NC * N, D), f32),
                  jax.ShapeDtypeStruct((NC * N, 16), f32)),
        mesh=_sc_mesh,
        scratch_types=[pltpu.VMEM((2, CH), i32),
                       pltpu.VMEM((2, CH, D), f32),
                       pltpu.VMEM((CH, 16), f32),
                       pltpu.VMEM((ZR, D), f32),
                       pltpu.VMEM((ZR, 16), f32),
                       pltpu.VMEM_SHARED((N, D), f32),
                       pltpu.VMEM_SHARED((N, 16), f32),
                       pltpu.SemaphoreType.DMA,
                       pltpu.SemaphoreType.DMA],
        compiler_params=_sc_params)
    def sk(msg_hbm, dst_hbm, agg_hbm, deg_hbm,
           idx_v, rows_v, ones_v, zb_v, zb16_v, agg_sh, deg_sh,
           seml0, seml1):
        c = lax.axis_index("c")
        s = lax.axis_index("s")
        seml = (seml0, seml1)

        @pl.loop(0, CH)
        def _(i):
            ones_v.at[pl.ds(i, 1), :][...] = jnp.ones((1, 16), f32)

        @pl.loop(0, ZR)
        def _(i):
            zb16_v.at[pl.ds(i, 1), :][...] = jnp.zeros((1, 16), f32)

            @pl.loop(0, D, step=16)
            def _(j):
                zb_v.at[pl.ds(i, 1), pl.ds(j, 16)][...] = jnp.zeros((1, 16), f32)

        @pl.loop(0, STR, step=ZR)
        def _(k):
            pltpu.sync_copy(zb_v, agg_sh.at[pl.ds(s * STR + k, ZR)])
            pltpu.sync_copy(zb16_v, deg_sh.at[pl.ds(s * STR + k, ZR)])

        plsc.subcore_barrier()
        wbase = c * (SLAB // NC) + s * EPW

        def issue_load(w, b):
            off = wbase + w * CH
            pltpu.async_copy(dst_hbm.at[pl.ds(off, CH)], idx_v.at[b], seml[b])
            pltpu.async_copy(msg_hbm.at[pl.ds(off, CH)], rows_v.at[b], seml[b])

        def wait_load(w, b):
            off = wbase + w * CH
            pltpu.make_async_copy(dst_hbm.at[pl.ds(off, CH)], idx_v.at[b], seml[b]).wait()
            pltpu.make_async_copy(msg_hbm.at[pl.ds(off, CH)], rows_v.at[b], seml[b]).wait()

        issue_load(0, 0)
        issue_load(1, 1)

        @pl.loop(0, (STEPS + 1) // 2)
        def _(g):
            for b in (0, 1):
                w = 2 * g + b

                @pl.when(w < STEPS)
                def _():
                    wait_load(w, b)
                    pltpu.sync_copy(rows_v.at[b], agg_sh.at[idx_v.at[b]], add=True)
                    pltpu.sync_copy(ones_v, deg_sh.at[idx_v.at[b]], add=True)

                    @pl.when(w + 2 < STEPS)
                    def _():
                        issue_load(w + 2, b)

        plsc.subcore_barrier()

        @pl.loop(0, STR, step=ZR)
        def _(k):
            off = s * STR + k
            pltpu.sync_copy(agg_sh.at[pl.ds(off, ZR)], zb_v)
            pltpu.sync_copy(zb_v, agg_hbm.at[pl.ds(c * N + off, ZR)])
            pltpu.sync_copy(deg_sh.at[pl.ds(off, ZR)], zb16_v)
            pltpu.sync_copy(zb16_v, deg_hbm.at[pl.ds(c * N + off, ZR)])

    return sk(msg, dst)


def _node_body(x_ref, a0_ref, a1_ref, a2_ref, a3_ref, a4_ref, a5_ref,
               d0_ref, d1_ref, d2_ref, d3_ref, d4_ref, d5_ref,
               wa_ref, wb_ref, bg2_ref,
               wo1_ref, bo1_ref, wobj_ref, bobj_ref, nc_ref):
    agg = (a0_ref[...] + a1_ref[...] + a2_ref[...] + a3_ref[...]
           + a4_ref[...] + a5_ref[...])
    deg = jnp.max(d0_ref[...] + d1_ref[...] + d2_ref[...] + d3_ref[...]
                  + d4_ref[...] + d5_ref[...],
                  axis=1, keepdims=True)
    agg = agg / jnp.maximum(deg, 1.0)
    xn = jax.nn.relu(jnp.dot(x_ref[...], wa_ref[...], preferred_element_type=f32)
                     + jnp.dot(agg, wb_ref[...], preferred_element_type=f32)
                     + bg2_ref[...])
    ho = jax.nn.relu(jnp.dot(xn, wo1_ref[...], preferred_element_type=f32)
                     + bo1_ref[...])
    nc_ref[...] = jnp.dot(ho, wobj_ref[...], preferred_element_type=f32) \
        + bobj_ref[...]


def _node_stage(x, aggs, degs, wg2a, wg2b, bg2r, Wo1, bo1r, Wobj, bobjr):
    full = lambda a, b: pl.BlockSpec((a, b), lambda i: (0, 0))
    blk = lambda w: pl.BlockSpec((BN, w), lambda i: (i, 0))
    return pl.pallas_call(
        _node_body,
        grid=(N // BN,),
        in_specs=[blk(D), blk(D), blk(D), blk(D), blk(D), blk(D), blk(D),
                  blk(16), blk(16), blk(16), blk(16), blk(16), blk(16),
                  full(D, D), full(D, D), full(1, D),
                  full(D, D), full(1, D),
                  full(D, 160), full(1, 160)],
        out_specs=[pl.BlockSpec((BN, 160), lambda i: (i, 0))],
        out_shape=[jax.ShapeDtypeStruct((N, 160), f32)],
    )(x, *aggs, *degs, wg2a, wg2b, bg2r, Wo1, bo1r, Wobj, bobjr)[0]


def kernel(x, descriptor, node_edges, We1, be1, We2, be2, Wg1, bg1, Wg2, bg2,
           Wo1, bo1, Wobj, bobj, Wr1, br1, Wrel, brel):
    src = node_edges[0].astype(i32)
    dst = node_edges[1].astype(i32)
    desc16 = jnp.pad(descriptor, ((0, 0), (0, 5)))
    wg1a, wg1m, wg1c = Wg1[:D], Wg1[D:2 * D], Wg1[2 * D:]
    w1p = jnp.zeros((32, 64), f32).at[0:11].set(We1[:11]).at[16:27].set(We1[11:])

    ts, td = _build_tables(
        x, desc16, wg1a[:, :64], wg1a[:, 64:], wg1c[:, :64], wg1c[:, 64:],
        bg1[:64].reshape(1, 64), bg1[64:].reshape(1, 64))

    ecs, aggs, degs = [], [], []
    for off, nsl in SLABS:
        srcs = lax.dynamic_slice_in_dim(src, off, nsl)
        dsts = lax.dynamic_slice_in_dim(dst, off, nsl)
        gs, gd = _gather_stage(ts, td, srcs, dsts)
        msg, ec = _edge_stage(gs, gd, w1p, be1.reshape(1, 64),
                              We2, be2.reshape(1, D), wg1m, Wr1,
                              br1.reshape(1, D), Wrel, brel.reshape(1, 26))
        ecs.append(ec)
        agg2, deg2 = _scatter_stage(msg, dsts)
        aggs.extend([agg2[:N], agg2[N:]])
        degs.extend([deg2[:N], deg2[N:]])
    ecls = jnp.concatenate(ecs, axis=0)
    ncls = _node_stage(x, aggs, degs, Wg2[:D], Wg2[D:],
                       bg2.reshape(1, D), Wo1, bo1.reshape(1, D),
                       Wobj, bobj.reshape(1, 160))
    return (ncls, ecls)


# speedup vs baseline: 1.0459x; 1.0459x over previous
import functools

import jax
import jax.numpy as jnp
from jax import lax
from jax.experimental import pallas as pl
from jax.experimental.pallas import tpu as pltpu
from jax.experimental.pallas import tpu_sc as plsc

N = 10000
E = 320000
D = 128
NC, NS = 2, 16
NWK = NC * NS
GW = 80
CH = 80
BN = 2000
BE = 2560
SLABS = ((0, 163840), (163840, 156160))

f32 = jnp.float32
i32 = jnp.int32
u32 = jnp.uint32

_sc_mesh = plsc.VectorSubcoreMesh(
    core_axis_name="c", subcore_axis_name="s", num_cores=NC, num_subcores=NS)
_sc_params = pltpu.CompilerParams(use_tc_tiling_on_sc=False)


def _pack2(lo_f, hi_f):
    lo_u = lax.bitcast_convert_type(lo_f, u32) + u32(0x8000)
    hi_u = lax.bitcast_convert_type(hi_f, u32) + u32(0x8000)
    w = lax.bitwise_or(lax.shift_right_logical(lo_u, u32(16)),
                       lax.bitwise_and(hi_u, u32(0xFFFF0000)))
    return lax.bitcast_convert_type(w, i32)


def _unpack2(w_i32):
    w = lax.bitcast_convert_type(w_i32, u32)
    lo = lax.bitcast_convert_type(lax.shift_left(w, u32(16)), f32)
    hi = lax.bitcast_convert_type(lax.bitwise_and(w, u32(0xFFFF0000)), f32)
    return lo, hi


def _tables_body(x_ref, d_ref, wal_ref, wah_ref, wcl_ref, wch_ref,
                 bl_ref, bh_ref, ts_ref, td_ref):
    xb = x_ref[...]
    db = d_ref[...]
    dp = _pack2(db[:, :8], db[:, 8:16])
    z = jnp.zeros((xb.shape[0], 8), i32)
    ps_lo = jnp.dot(xb, wal_ref[...], preferred_element_type=f32) + bl_ref[...]
    ps_hi = jnp.dot(xb, wah_ref[...], preferred_element_type=f32) + bh_ref[...]
    pd_lo = jnp.dot(xb, wcl_ref[...], preferred_element_type=f32)
    pd_hi = jnp.dot(xb, wch_ref[...], preferred_element_type=f32)
    ts_ref[...] = jnp.concatenate([_pack2(ps_lo, ps_hi), dp, z], axis=1)
    td_ref[...] = jnp.concatenate([_pack2(pd_lo, pd_hi), dp, z], axis=1)


def _build_tables(x, desc16, wal, wah, wcl, wch, bl, bh):
    full = lambda a, b: pl.BlockSpec((a, b), lambda i: (0, 0))
    return pl.pallas_call(
        _tables_body,
        grid=(N // BN,),
        in_specs=[pl.BlockSpec((BN, D), lambda i: (i, 0)),
                  pl.BlockSpec((BN, 16), lambda i: (i, 0)),
                  full(D, 64), full(D, 64), full(D, 64), full(D, 64),
                  full(1, 64), full(1, 64)],
        out_specs=[pl.BlockSpec((BN, 80), lambda i: (i, 0)),
                   pl.BlockSpec((BN, 80), lambda i: (i, 0))],
        out_shape=[jax.ShapeDtypeStruct((N, 80), i32),
                   jax.ShapeDtypeStruct((N, 80), i32)],
    )(x, desc16, wal, wah, wcl, wch, bl, bh)


def _gather_stage(ts, td, src, dst):
    SLAB = src.shape[0]
    EPW = SLAB // NWK
    STEPS = EPW // GW

    @functools.partial(
        pl.kernel,
        out_type=(jax.ShapeDtypeStruct((SLAB, D), i32),
                  jax.ShapeDtypeStruct((SLAB, D), i32)),
        mesh=_sc_mesh,
        scratch_types=[pltpu.VMEM((2, GW), i32),
                       pltpu.VMEM((2, GW), i32),
                       pltpu.VMEM((2, GW, 80), i32),
                       pltpu.VMEM((2, GW, 80), i32),
                       pltpu.SemaphoreType.DMA,
                       pltpu.SemaphoreType.DMA,
                       pltpu.SemaphoreType.DMA,
                       pltpu.SemaphoreType.DMA,
                       pltpu.SemaphoreType.DMA,
                       pltpu.SemaphoreType.DMA],
        compiler_params=_sc_params)
    def gk(ts_hbm, td_hbm, src_hbm, dst_hbm,
           gs_hbm, gd_hbm,
           is_v, id_v, gs_v, gd_v,
           semi0, semi1, semg0, semg1, semo0, semo1):
        wid = lax.axis_index("s") * NC + lax.axis_index("c")
        base = wid * EPW
        semi = (semi0, semi1)
        semg = (semg0, semg1)
        semo = (semo0, semo1)

        def issue_idx(w, b):
            off = base + w * GW
            pltpu.async_copy(src_hbm.at[pl.ds(off, GW)], is_v.at[b], semi[b])
            pltpu.async_copy(dst_hbm.at[pl.ds(off, GW)], id_v.at[b], semi[b])

        def wait_idx(w, b):
            off = base + w * GW
            pltpu.make_async_copy(src_hbm.at[pl.ds(off, GW)], is_v.at[b], semi[b]).wait()
            pltpu.make_async_copy(dst_hbm.at[pl.ds(off, GW)], id_v.at[b], semi[b]).wait()

        def issue_gather(b):
            pltpu.async_copy(ts_hbm.at[is_v.at[b]], gs_v.at[b], semg[b])
            pltpu.async_copy(td_hbm.at[id_v.at[b]], gd_v.at[b], semg[b])

        def wait_gather(b):
            pltpu.make_async_copy(ts_hbm.at[is_v.at[b]], gs_v.at[b], semg[b]).wait()
            pltpu.make_async_copy(td_hbm.at[id_v.at[b]], gd_v.at[b], semg[b]).wait()

        def issue_out(w, b):
            off = base + w * GW
            pltpu.async_copy(gs_v.at[b], gs_hbm.at[pl.ds(off, GW), pl.ds(0, 80)], semo[b])
            pltpu.async_copy(gd_v.at[b], gd_hbm.at[pl.ds(off, GW), pl.ds(0, 80)], semo[b])

        def wait_out(w, b):
            off = base + w * GW
            pltpu.make_async_copy(gs_v.at[b], gs_hbm.at[pl.ds(off, GW), pl.ds(0, 80)], semo[b]).wait()
            pltpu.make_async_copy(gd_v.at[b], gd_hbm.at[pl.ds(off, GW), pl.ds(0, 80)], semo[b]).wait()

        issue_idx(0, 0)
        issue_idx(1, 1)
        wait_idx(0, 0)
        issue_gather(0)
        wait_idx(1, 1)
        issue_gather(1)

        @pl.loop(0, (STEPS + 1) // 2)
        def _(g):
            for b in (0, 1):
                w = 2 * g + b

                @pl.when(w < STEPS)
                def _():
                    wait_gather(b)
                    issue_out(w, b)

                    @pl.when(w + 2 < STEPS)
                    def _():
                        issue_idx(w + 2, b)
                        wait_out(w, b)
                        wait_idx(w + 2, b)
                        issue_gather(b)

        wait_out(STEPS - 2, (STEPS - 2) % 2)
        wait_out(STEPS - 1, (STEPS - 1) % 2)

    return gk(ts, td, src, dst)


def _edge_body(gs_ref, gd_ref, w1_ref, b1_ref, w2_ref,
               b2_ref, wm_ref, wr1_ref, br1_ref,
               wrel_ref, brel_ref, msg_ref, ec_ref):
    s_lo, s_hi = _unpack2(gs_ref[...])
    d_lo, d_hi = _unpack2(gd_ref[...])
    dcat = jnp.concatenate([s_lo[:, 64:72], s_hi[:, 64:72],
                            d_lo[:, 64:72], d_hi[:, 64:72]], axis=1)
    h = jax.nn.relu(jnp.dot(dcat, w1_ref[...], preferred_element_type=f32)
                    + b1_ref[...])
    e = jax.nn.relu(jnp.dot(h, w2_ref[...], preferred_element_type=f32)
                    + b2_ref[...])
    p = jnp.concatenate([s_lo[:, :64] + d_lo[:, :64],
                         s_hi[:, :64] + d_hi[:, :64]], axis=1)
    m = jax.nn.relu(p + jnp.dot(e, wm_ref[...], preferred_element_type=f32))
    msg_ref[...] = m
    hr = jax.nn.relu(jnp.dot(m, wr1_ref[...], preferred_element_type=f32)
                     + br1_ref[...])
    ec_ref[...] = jnp.dot(hr, wrel_ref[...], preferred_element_type=f32) \
        + brel_ref[...]


def _edge_stage(gs, gd, w1p, be1r, We2, be2r, wg1m, Wr1, br1r,
                Wrel, brelr):
    SLAB = gs.shape[0]
    full = lambda a, b: pl.BlockSpec((a, b), lambda i: (0, 0))
    return pl.pallas_call(
        _edge_body,
        grid=(SLAB // BE,),
        in_specs=[pl.BlockSpec((BE, D), lambda i: (i, 0)),
                  pl.BlockSpec((BE, D), lambda i: (i, 0)),
                  full(32, 64), full(1, 64),
                  full(64, D), full(1, D),
                  full(D, D),
                  full(D, D), full(1, D),
                  full(D, 26), full(1, 26)],
        out_specs=[pl.BlockSpec((BE, D), lambda i: (i, 0)),
                   pl.BlockSpec((BE, 26), lambda i: (i, 0))],
        out_shape=[jax.ShapeDtypeStruct((SLAB, D), f32),
                   jax.ShapeDtypeStruct((SLAB, 26), f32)],
    )(gs, gd, w1p, be1r, We2, be2r, wg1m, Wr1, br1r, Wrel, brelr)


def _scatter_stage(msg, dst):
    SLAB = msg.shape[0]
    EPW = SLAB // NWK
    STEPS = EPW // CH
    STR = N // NS
    ZR = 125

    @functools.partial(
        pl.kernel,
        out_type=(jax.ShapeDtypeStruct((NC * N, D), f32),
                  jax.ShapeDtypeStruct((NC * N, 16), f32)),
        mesh=_sc_mesh,
        scratch_types=[pltpu.VMEM((2, CH), i32),
                       pltpu.VMEM((2, CH, D), f32),
                       pltpu.VMEM((CH, 16), f32),
                       pltpu.VMEM((ZR, D), f32),
                       pltpu.VMEM((ZR, 16), f32),
                       pltpu.VMEM_SHARED((N, D), f32),
                       pltpu.VMEM_SHARED((N, 16), f32),
                       pltpu.SemaphoreType.DMA,
                       pltpu.SemaphoreType.DMA],
        compiler_params=_sc_params)
    def sk(msg_hbm, dst_hbm, agg_hbm, deg_hbm,
           idx_v, rows_v, ones_v, zb_v, zb16_v, agg_sh, deg_sh,
           seml0, seml1):
        c = lax.axis_index("c")
        s = lax.axis_index("s")
        seml = (seml0, seml1)

        @pl.loop(0, CH)
        def _(i):
            ones_v.at[pl.ds(i, 1), :][...] = jnp.ones((1, 16), f32)

        @pl.loop(0, ZR)
        def _(i):
            zb16_v.at[pl.ds(i, 1), :][...] = jnp.zeros((1, 16), f32)

            @pl.loop(0, D, step=16)
            def _(j):
                zb_v.at[pl.ds(i, 1), pl.ds(j, 16)][...] = jnp.zeros((1, 16), f32)

        @pl.loop(0, STR, step=ZR)
        def _(k):
            pltpu.sync_copy(zb_v, agg_sh.at[pl.ds(s * STR + k, ZR)])
            pltpu.sync_copy(zb16_v, deg_sh.at[pl.ds(s * STR + k, ZR)])

        plsc.subcore_barrier()
        wbase = c * (SLAB // NC) + s * EPW

        def issue_load(w, b):
            off = wbase + w * CH
            pltpu.async_copy(dst_hbm.at[pl.ds(off, CH)], idx_v.at[b], seml[b])
            pltpu.async_copy(msg_hbm.at[pl.ds(off, CH)], rows_v.at[b], seml[b])

        def wait_load(w, b):
            off = wbase + w * CH
            pltpu.make_async_copy(dst_hbm.at[pl.ds(off, CH)], idx_v.at[b], seml[b]).wait()
            pltpu.make_async_copy(msg_hbm.at[pl.ds(off, CH)], rows_v.at[b], seml[b]).wait()

        issue_load(0, 0)
        issue_load(1, 1)

        @pl.loop(0, (STEPS + 1) // 2)
        def _(g):
            for b in (0, 1):
                w = 2 * g + b

                @pl.when(w < STEPS)
                def _():
                    wait_load(w, b)
                    pltpu.sync_copy(rows_v.at[b], agg_sh.at[idx_v.at[b]], add=True)
                    pltpu.sync_copy(ones_v, deg_sh.at[idx_v.at[b]], add=True)

                    @pl.when(w + 2 < STEPS)
                    def _():
                        issue_load(w + 2, b)

        plsc.subcore_barrier()

        @pl.loop(0, STR, step=ZR)
        def _(k):
            off = s * STR + k
            pltpu.sync_copy(agg_sh.at[pl.ds(off, ZR)], zb_v)
            pltpu.sync_copy(zb_v, agg_hbm.at[pl.ds(c * N + off, ZR)])
            pltpu.sync_copy(deg_sh.at[pl.ds(off, ZR)], zb16_v)
            pltpu.sync_copy(zb16_v, deg_hbm.at[pl.ds(c * N + off, ZR)])

    return sk(msg, dst)


def _node_body(x_ref, a0_ref, a1_ref, a2_ref, a3_ref,
               d0_ref, d1_ref, d2_ref, d3_ref, wa_ref, wb_ref, bg2_ref,
               wo1_ref, bo1_ref, wobj_ref, bobj_ref, nc_ref):
    agg = a0_ref[...] + a1_ref[...] + a2_ref[...] + a3_ref[...]
    deg = jnp.max(d0_ref[...] + d1_ref[...] + d2_ref[...] + d3_ref[...],
                  axis=1, keepdims=True)
    agg = agg / jnp.maximum(deg, 1.0)
    xn = jax.nn.relu(jnp.dot(x_ref[...], wa_ref[...], preferred_element_type=f32)
                     + jnp.dot(agg, wb_ref[...], preferred_element_type=f32)
                     + bg2_ref[...])
    ho = jax.nn.relu(jnp.dot(xn, wo1_ref[...], preferred_element_type=f32)
                     + bo1_ref[...])
    nc_ref[...] = jnp.dot(ho, wobj_ref[...], preferred_element_type=f32) \
        + bobj_ref[...]


def _node_stage(x, aggs, degs, wg2a, wg2b, bg2r, Wo1, bo1r, Wobj, bobjr):
    full = lambda a, b: pl.BlockSpec((a, b), lambda i: (0, 0))
    blk = lambda w: pl.BlockSpec((BN, w), lambda i: (i, 0))
    return pl.pallas_call(
        _node_body,
        grid=(N // BN,),
        in_specs=[blk(D), blk(D), blk(D), blk(D), blk(D),
                  blk(16), blk(16), blk(16), blk(16),
                  full(D, D), full(D, D), full(1, D),
                  full(D, D), full(1, D),
                  full(D, 160), full(1, 160)],
        out_specs=[pl.BlockSpec((BN, 160), lambda i: (i, 0))],
        out_shape=[jax.ShapeDtypeStruct((N, 160), f32)],
    )(x, *aggs, *degs, wg2a, wg2b, bg2r, Wo1, bo1r, Wobj, bobjr)[0]


def kernel(x, descriptor, node_edges, We1, be1, We2, be2, Wg1, bg1, Wg2, bg2,
           Wo1, bo1, Wobj, bobj, Wr1, br1, Wrel, brel):
    src = node_edges[0].astype(i32)
    dst = node_edges[1].astype(i32)
    desc16 = jnp.pad(descriptor, ((0, 0), (0, 5)))
    wg1a, wg1m, wg1c = Wg1[:D], Wg1[D:2 * D], Wg1[2 * D:]
    w1p = jnp.zeros((32, 64), f32).at[0:11].set(We1[:11]).at[16:27].set(We1[11:])

    ts, td = _build_tables(
        x, desc16, wg1a[:, :64], wg1a[:, 64:], wg1c[:, :64], wg1c[:, 64:],
        bg1[:64].reshape(1, 64), bg1[64:].reshape(1, 64))

    ecs, aggs, degs = [], [], []
    for off, nsl in SLABS:
        srcs = lax.dynamic_slice_in_dim(src, off, nsl)
        dsts = lax.dynamic_slice_in_dim(dst, off, nsl)
        gs, gd = _gather_stage(ts, td, srcs, dsts)
        msg, ec = _edge_stage(gs, gd, w1p, be1.reshape(1, 64),
                              We2, be2.reshape(1, D), wg1m, Wr1,
                              br1.reshape(1, D), Wrel, brel.reshape(1, 26))
        ecs.append(ec)
        agg2, deg2 = _scatter_stage(msg, dsts)
        aggs.extend([agg2[:N], agg2[N:]])
        degs.extend([deg2[:N], deg2[N:]])
    ecls = jnp.concatenate(ecs, axis=0)
    ncls = _node_stage(x, aggs, degs, Wg2[:D], Wg2[D:],
                       bg2.reshape(1, D), Wo1, bo1.reshape(1, D),
                       Wobj, bobj.reshape(1, 160))
    return (ncls, ecls)


# speedup vs baseline: 1.0466x; 1.0007x over previous
import functools

import jax
import jax.numpy as jnp
from jax import lax
from jax.experimental import pallas as pl
from jax.experimental.pallas import tpu as pltpu
from jax.experimental.pallas import tpu_sc as plsc

N = 10000
E = 320000
D = 128
NC, NS = 2, 16
NWK = NC * NS
GW = 80
CH = 80
BN = 2000
BE = 2560
SLABS = ((0, 215040), (215040, 104960))

f32 = jnp.float32
i32 = jnp.int32
u32 = jnp.uint32

_sc_mesh = plsc.VectorSubcoreMesh(
    core_axis_name="c", subcore_axis_name="s", num_cores=NC, num_subcores=NS)
_sc_params = pltpu.CompilerParams(use_tc_tiling_on_sc=False)


def _pack2(lo_f, hi_f):
    lo_u = lax.bitcast_convert_type(lo_f, u32) + u32(0x8000)
    hi_u = lax.bitcast_convert_type(hi_f, u32) + u32(0x8000)
    w = lax.bitwise_or(lax.shift_right_logical(lo_u, u32(16)),
                       lax.bitwise_and(hi_u, u32(0xFFFF0000)))
    return lax.bitcast_convert_type(w, i32)


def _unpack2(w_i32):
    w = lax.bitcast_convert_type(w_i32, u32)
    lo = lax.bitcast_convert_type(lax.shift_left(w, u32(16)), f32)
    hi = lax.bitcast_convert_type(lax.bitwise_and(w, u32(0xFFFF0000)), f32)
    return lo, hi


def _tables_body(x_ref, d_ref, wal_ref, wah_ref, wcl_ref, wch_ref,
                 bl_ref, bh_ref, ts_ref, td_ref):
    xb = x_ref[...]
    db = d_ref[...]
    dp = _pack2(db[:, :8], db[:, 8:16])
    z = jnp.zeros((xb.shape[0], 8), i32)
    ps_lo = jnp.dot(xb, wal_ref[...], preferred_element_type=f32) + bl_ref[...]
    ps_hi = jnp.dot(xb, wah_ref[...], preferred_element_type=f32) + bh_ref[...]
    pd_lo = jnp.dot(xb, wcl_ref[...], preferred_element_type=f32)
    pd_hi = jnp.dot(xb, wch_ref[...], preferred_element_type=f32)
    ts_ref[...] = jnp.concatenate([_pack2(ps_lo, ps_hi), dp, z], axis=1)
    td_ref[...] = jnp.concatenate([_pack2(pd_lo, pd_hi), dp, z], axis=1)


def _build_tables(x, desc16, wal, wah, wcl, wch, bl, bh):
    full = lambda a, b: pl.BlockSpec((a, b), lambda i: (0, 0))
    return pl.pallas_call(
        _tables_body,
        grid=(N // BN,),
        in_specs=[pl.BlockSpec((BN, D), lambda i: (i, 0)),
                  pl.BlockSpec((BN, 16), lambda i: (i, 0)),
                  full(D, 64), full(D, 64), full(D, 64), full(D, 64),
                  full(1, 64), full(1, 64)],
        out_specs=[pl.BlockSpec((BN, 80), lambda i: (i, 0)),
                   pl.BlockSpec((BN, 80), lambda i: (i, 0))],
        out_shape=[jax.ShapeDtypeStruct((N, 80), i32),
                   jax.ShapeDtypeStruct((N, 80), i32)],
    )(x, desc16, wal, wah, wcl, wch, bl, bh)


def _gather_stage(ts, td, src, dst):
    SLAB = src.shape[0]
    EPW = SLAB // NWK
    STEPS = EPW // GW

    @functools.partial(
        pl.kernel,
        out_type=(jax.ShapeDtypeStruct((SLAB, D), i32),
                  jax.ShapeDtypeStruct((SLAB, D), i32)),
        mesh=_sc_mesh,
        scratch_types=[pltpu.VMEM((2, GW), i32),
                       pltpu.VMEM((2, GW), i32),
                       pltpu.VMEM((2, GW, 80), i32),
                       pltpu.VMEM((2, GW, 80), i32),
                       pltpu.SemaphoreType.DMA,
                       pltpu.SemaphoreType.DMA,
                       pltpu.SemaphoreType.DMA,
                       pltpu.SemaphoreType.DMA,
                       pltpu.SemaphoreType.DMA,
                       pltpu.SemaphoreType.DMA],
        compiler_params=_sc_params)
    def gk(ts_hbm, td_hbm, src_hbm, dst_hbm,
           gs_hbm, gd_hbm,
           is_v, id_v, gs_v, gd_v,
           semi0, semi1, semg0, semg1, semo0, semo1):
        wid = lax.axis_index("s") * NC + lax.axis_index("c")
        base = wid * EPW
        semi = (semi0, semi1)
        semg = (semg0, semg1)
        semo = (semo0, semo1)

        def issue_idx(w, b):
            off = base + w * GW
            pltpu.async_copy(src_hbm.at[pl.ds(off, GW)], is_v.at[b], semi[b])
            pltpu.async_copy(dst_hbm.at[pl.ds(off, GW)], id_v.at[b], semi[b])

        def wait_idx(w, b):
            off = base + w * GW
            pltpu.make_async_copy(src_hbm.at[pl.ds(off, GW)], is_v.at[b], semi[b]).wait()
            pltpu.make_async_copy(dst_hbm.at[pl.ds(off, GW)], id_v.at[b], semi[b]).wait()

        def issue_gather(b):
            pltpu.async_copy(ts_hbm.at[is_v.at[b]], gs_v.at[b], semg[b])
            pltpu.async_copy(td_hbm.at[id_v.at[b]], gd_v.at[b], semg[b])

        def wait_gather(b):
            pltpu.make_async_copy(ts_hbm.at[is_v.at[b]], gs_v.at[b], semg[b]).wait()
            pltpu.make_async_copy(td_hbm.at[id_v.at[b]], gd_v.at[b], semg[b]).wait()

        def issue_out(w, b):
            off = base + w * GW
            pltpu.async_copy(gs_v.at[b], gs_hbm.at[pl.ds(off, GW), pl.ds(0, 80)], semo[b])
            pltpu.async_copy(gd_v.at[b], gd_hbm.at[pl.ds(off, GW), pl.ds(0, 80)], semo[b])

        def wait_out(w, b):
            off = base + w * GW
            pltpu.make_async_copy(gs_v.at[b], gs_hbm.at[pl.ds(off, GW), pl.ds(0, 80)], semo[b]).wait()
            pltpu.make_async_copy(gd_v.at[b], gd_hbm.at[pl.ds(off, GW), pl.ds(0, 80)], semo[b]).wait()

        issue_idx(0, 0)
        issue_idx(1, 1)
        wait_idx(0, 0)
        issue_gather(0)
        wait_idx(1, 1)
        issue_gather(1)

        @pl.loop(0, (STEPS + 1) // 2)
        def _(g):
            for b in (0, 1):
                w = 2 * g + b

                @pl.when(w < STEPS)
                def _():
                    wait_gather(b)
                    issue_out(w, b)

                    @pl.when(w + 2 < STEPS)
                    def _():
                        issue_idx(w + 2, b)
                        wait_out(w, b)
                        wait_idx(w + 2, b)
                        issue_gather(b)

        wait_out(STEPS - 2, (STEPS - 2) % 2)
        wait_out(STEPS - 1, (STEPS - 1) % 2)

    return gk(ts, td, src, dst)


def _edge_body(gs_ref, gd_ref, w1_ref, b1_ref, w2_ref,
               b2_ref, wm_ref, wr1_ref, br1_ref,
               wrel_ref, brel_ref, msg_ref, ec_ref):
    s_lo, s_hi = _unpack2(gs_ref[...])
    d_lo, d_hi = _unpack2(gd_ref[...])
    dcat = jnp.concatenate([s_lo[:, 64:72], s_hi[:, 64:72],
                            d_lo[:, 64:72], d_hi[:, 64:72]], axis=1)
    h = jax.nn.relu(jnp.dot(dcat, w1_ref[...], preferred_element_type=f32)
                    + b1_ref[...])
    e = jax.nn.relu(jnp.dot(h, w2_ref[...], preferred_element_type=f32)
                    + b2_ref[...])
    p = jnp.concatenate([s_lo[:, :64] + d_lo[:, :64],
                         s_hi[:, :64] + d_hi[:, :64]], axis=1)
    m = jax.nn.relu(p + jnp.dot(e, wm_ref[...], preferred_element_type=f32))
    msg_ref[...] = m
    hr = jax.nn.relu(jnp.dot(m, wr1_ref[...], preferred_element_type=f32)
                     + br1_ref[...])
    ec_ref[...] = jnp.dot(hr, wrel_ref[...], preferred_element_type=f32) \
        + brel_ref[...]


def _edge_stage(gs, gd, w1p, be1r, We2, be2r, wg1m, Wr1, br1r,
                Wrel, brelr):
    SLAB = gs.shape[0]
    full = lambda a, b: pl.BlockSpec((a, b), lambda i: (0, 0))
    return pl.pallas_call(
        _edge_body,
        grid=(SLAB // BE,),
        in_specs=[pl.BlockSpec((BE, D), lambda i: (i, 0)),
                  pl.BlockSpec((BE, D), lambda i: (i, 0)),
                  full(32, 64), full(1, 64),
                  full(64, D), full(1, D),
                  full(D, D),
                  full(D, D), full(1, D),
                  full(D, 26), full(1, 26)],
        out_specs=[pl.BlockSpec((BE, D), lambda i: (i, 0)),
                   pl.BlockSpec((BE, 26), lambda i: (i, 0))],
        out_shape=[jax.ShapeDtypeStruct((SLAB, D), f32),
                   jax.ShapeDtypeStruct((SLAB, 26), f32)],
    )(gs, gd, w1p, be1r, We2, be2r, wg1m, Wr1, br1r, Wrel, brelr)


def _scatter_stage(msg, dst):
    SLAB = msg.shape[0]
    EPW = SLAB // NWK
    STEPS = EPW // CH
    STR = N // NS
    ZR = 125

    @functools.partial(
        pl.kernel,
        out_type=(jax.ShapeDtypeStruct((NC * N, D), f32),
                  jax.ShapeDtypeStruct((NC * N, 16), f32)),
        mesh=_sc_mesh,
        scratch_types=[pltpu.VMEM((2, CH), i32),
                       pltpu.VMEM((2, CH, D), f32),
                       pltpu.VMEM((CH, 16), f32),
                       pltpu.VMEM((ZR, D), f32),
                       pltpu.VMEM((ZR, 16), f32),
                       pltpu.VMEM_SHARED((N, D), f32),
                       pltpu.VMEM_SHARED((N, 16), f32),
                       pltpu.SemaphoreType.DMA,
                       pltpu.SemaphoreType.DMA],
        compiler_params=_sc_params)
    def sk(msg_hbm, dst_hbm, agg_hbm, deg_hbm,
           idx_v, rows_v, ones_v, zb_v, zb16_v, agg_sh, deg_sh,
           seml0, seml1):
        c = lax.axis_index("c")
        s = lax.axis_index("s")
        seml = (seml0, seml1)

        @pl.loop(0, CH)
        def _(i):
            ones_v.at[pl.ds(i, 1), :][...] = jnp.ones((1, 16), f32)

        @pl.loop(0, ZR)
        def _(i):
            zb16_v.at[pl.ds(i, 1), :][...] = jnp.zeros((1, 16), f32)

            @pl.loop(0, D, step=16)
            def _(j):
                zb_v.at[pl.ds(i, 1), pl.ds(j, 16)][...] = jnp.zeros((1, 16), f32)

        @pl.loop(0, STR, step=ZR)
        def _(k):
            pltpu.sync_copy(zb_v, agg_sh.at[pl.ds(s * STR + k, ZR)])
            pltpu.sync_copy(zb16_v, deg_sh.at[pl.ds(s * STR + k, ZR)])

        plsc.subcore_barrier()
        wbase = c * (SLAB // NC) + s * EPW

        def issue_load(w, b):
            off = wbase + w * CH
            pltpu.async_copy(dst_hbm.at[pl.ds(off, CH)], idx_v.at[b], seml[b])
            pltpu.async_copy(msg_hbm.at[pl.ds(off, CH)], rows_v.at[b], seml[b])

        def wait_load(w, b):
            off = wbase + w * CH
            pltpu.make_async_copy(dst_hbm.at[pl.ds(off, CH)], idx_v.at[b], seml[b]).wait()
            pltpu.make_async_copy(msg_hbm.at[pl.ds(off, CH)], rows_v.at[b], seml[b]).wait()

        issue_load(0, 0)
        issue_load(1, 1)

        @pl.loop(0, (STEPS + 1) // 2)
        def _(g):
            for b in (0, 1):
                w = 2 * g + b

                @pl.when(w < STEPS)
                def _():
                    wait_load(w, b)
                    pltpu.sync_copy(rows_v.at[b], agg_sh.at[idx_v.at[b]], add=True)
                    pltpu.sync_copy(ones_v, deg_sh.at[idx_v.at[b]], add=True)

                    @pl.when(w + 2 < STEPS)
                    def _():
                        issue_load(w + 2, b)

        plsc.subcore_barrier()

        @pl.loop(0, STR, step=ZR)
        def _(k):
            off = s * STR + k
            pltpu.sync_copy(agg_sh.at[pl.ds(off, ZR)], zb_v)
            pltpu.sync_copy(zb_v, agg_hbm.at[pl.ds(c * N + off, ZR)])
            pltpu.sync_copy(deg_sh.at[pl.ds(off, ZR)], zb16_v)
            pltpu.sync_copy(zb16_v, deg_hbm.at[pl.ds(c * N + off, ZR)])

    return sk(msg, dst)


def _node_body(x_ref, a0_ref, a1_ref, a2_ref, a3_ref,
               d0_ref, d1_ref, d2_ref, d3_ref, wa_ref, wb_ref, bg2_ref,
               wo1_ref, bo1_ref, wobj_ref, bobj_ref, nc_ref):
    agg = a0_ref[...] + a1_ref[...] + a2_ref[...] + a3_ref[...]
    deg = jnp.max(d0_ref[...] + d1_ref[...] + d2_ref[...] + d3_ref[...],
                  axis=1, keepdims=True)
    agg = agg / jnp.maximum(deg, 1.0)
    xn = jax.nn.relu(jnp.dot(x_ref[...], wa_ref[...], preferred_element_type=f32)
                     + jnp.dot(agg, wb_ref[...], preferred_element_type=f32)
                     + bg2_ref[...])
    ho = jax.nn.relu(jnp.dot(xn, wo1_ref[...], preferred_element_type=f32)
                     + bo1_ref[...])
    nc_ref[...] = jnp.dot(ho, wobj_ref[...], preferred_element_type=f32) \
        + bobj_ref[...]


def _node_stage(x, aggs, degs, wg2a, wg2b, bg2r, Wo1, bo1r, Wobj, bobjr):
    full = lambda a, b: pl.BlockSpec((a, b), lambda i: (0, 0))
    blk = lambda w: pl.BlockSpec((BN, w), lambda i: (i, 0))
    return pl.pallas_call(
        _node_body,
        grid=(N // BN,),
        in_specs=[blk(D), blk(D), blk(D), blk(D), blk(D),
                  blk(16), blk(16), blk(16), blk(16),
                  full(D, D), full(D, D), full(1, D),
                  full(D, D), full(1, D),
                  full(D, 160), full(1, 160)],
        out_specs=[pl.BlockSpec((BN, 160), lambda i: (i, 0))],
        out_shape=[jax.ShapeDtypeStruct((N, 160), f32)],
    )(x, *aggs, *degs, wg2a, wg2b, bg2r, Wo1, bo1r, Wobj, bobjr)[0]


def kernel(x, descriptor, node_edges, We1, be1, We2, be2, Wg1, bg1, Wg2, bg2,
           Wo1, bo1, Wobj, bobj, Wr1, br1, Wrel, brel):
    src = node_edges[0].astype(i32)
    dst = node_edges[1].astype(i32)
    desc16 = jnp.pad(descriptor, ((0, 0), (0, 5)))
    wg1a, wg1m, wg1c = Wg1[:D], Wg1[D:2 * D], Wg1[2 * D:]
    w1p = jnp.zeros((32, 64), f32).at[0:11].set(We1[:11]).at[16:27].set(We1[11:])

    ts, td = _build_tables(
        x, desc16, wg1a[:, :64], wg1a[:, 64:], wg1c[:, :64], wg1c[:, 64:],
        bg1[:64].reshape(1, 64), bg1[64:].reshape(1, 64))

    ecs, aggs, degs = [], [], []
    for off, nsl in SLABS:
        srcs = lax.dynamic_slice_in_dim(src, off, nsl)
        dsts = lax.dynamic_slice_in_dim(dst, off, nsl)
        gs, gd = _gather_stage(ts, td, srcs, dsts)
        msg, ec = _edge_stage(gs, gd, w1p, be1.reshape(1, 64),
                              We2, be2.reshape(1, D), wg1m, Wr1,
                              br1.reshape(1, D), Wrel, brel.reshape(1, 26))
        ecs.append(ec)
        agg2, deg2 = _scatter_stage(msg, dsts)
        aggs.extend([agg2[:N], agg2[N:]])
        degs.extend([deg2[:N], deg2[N:]])
    ecls = jnp.concatenate(ecs, axis=0)
    ncls = _node_stage(x, aggs, degs, Wg2[:D], Wg2[D:],
                       bg2.reshape(1, D), Wo1, bo1.reshape(1, D),
                       Wobj, bobj.reshape(1, 160))
    return (ncls, ecls)
